# baseline (speedup 1.0000x reference)
import functools
import math

import jax
import jax.numpy as jnp
from jax import lax
from jax.experimental import pallas as pl
from jax.experimental.pallas import tpu as pltpu
from jax.experimental.pallas import tpu_sc as plsc

_IDX_CHUNK = 128


@functools.lru_cache(maxsize=None)
def _make_gather(B, VP, DP, num_objs):
    info = plsc.get_sparse_core_info()
    NC, NS, L = info.num_cores, info.num_subcores, info.num_lanes
    NW = NC * NS
    D = DP // 2
    assert B % (8 * NW) == 0 and D % L == 0
    b_per_w = B // NW
    n_chunks = b_per_w // _IDX_CHUNK

    mesh = plsc.VectorSubcoreMesh(core_axis_name="c", subcore_axis_name="s")

    @functools.partial(
        pl.kernel,
        mesh=mesh,
        out_type=jax.ShapeDtypeStruct((B * D,), jnp.float32),
        compiler_params=pltpu.CompilerParams(use_tc_tiling_on_sc=True),
        scratch_types=[
            pltpu.VMEM((b_per_w,), jnp.int32),
            pltpu.VMEM((b_per_w,), jnp.int32),
            pltpu.VMEM((n_chunks, _IDX_CHUNK), jnp.int32),
            pltpu.VMEM((b_per_w,), jnp.int32),
            pltpu.VMEM((b_per_w, DP), jnp.float32),
            pltpu.VMEM((b_per_w * D,), jnp.float32),
            pltpu.SemaphoreType.DMA,
        ],
    )
    def gather_kernel(
        l0_hbm, l1_hbm, wp_hbm, out_hbm,
        l0_v, l1_v, jidx_v, half_v, prows_v, stage_v, sem,
    ):
        wid = lax.axis_index("s") * NC + lax.axis_index("c")
        base = wid * b_per_w
        pltpu.sync_copy(l0_hbm.at[pl.ds(base, b_per_w)], l0_v)
        pltpu.sync_copy(l1_hbm.at[pl.ds(base, b_per_w)], l1_v)
        for c in range(b_per_w // L):
            j, o = divmod(c * L, _IDX_CHUNK)
            r = l0_v[pl.ds(c * L, L)] * num_objs + l1_v[pl.ds(c * L, L)]
            jidx_v[j, pl.ds(o, L)] = r >> 1
            half_v[pl.ds(c * L, L)] = r & 1
        copies = [
            pltpu.async_copy(
                wp_hbm.at[jidx_v.at[j]],
                prows_v.at[pl.ds(j * _IDX_CHUNK, _IDX_CHUNK)],
                sem,
            )
            for j in range(n_chunks)
        ]
        for cp in copies:
            cp.wait()

        def extract(g, _):
            hvec = half_v[pl.ds(g * L, L)]
            for j in range(L):
                b = g * L + j
                h = jnp.full((L,), hvec[j], jnp.int32).astype(jnp.float32)
                nh = 1.0 - h
                for c in range(D // L):
                    lo = prows_v[b, pl.ds(c * L, L)]
                    hi = prows_v[b, pl.ds(D + c * L, L)]
                    stage_v[pl.ds(b * D + c * L, L)] = lo * nh + hi * h
            return _

        lax.fori_loop(0, b_per_w // L, extract, None)
        pltpu.sync_copy(stage_v, out_hbm.at[pl.ds(base * D, b_per_w * D)])

    return gather_kernel


def kernel(labels, W):
    B = labels.shape[0]
    V, D = W.shape
    num_objs = math.isqrt(V)
    wp = W.reshape(V // 2, 2 * D)
    l0 = labels[:, 0].astype(jnp.int32)
    l1 = labels[:, 1].astype(jnp.int32)
    flat = _make_gather(B, V // 2, 2 * D, num_objs)(l0, l1, wp)
    return flat.reshape(B, D)


# speedup vs baseline: 2.2842x; 2.2842x over previous
import functools
import math

import jax
import jax.numpy as jnp
from jax import lax
from jax.experimental import pallas as pl
from jax.experimental.pallas import tpu as pltpu
from jax.experimental.pallas import tpu_sc as plsc

_ROUND = 64


@functools.lru_cache(maxsize=None)
def _make_gather(B, NT, TR, D, num_objs):
    info = plsc.get_sparse_core_info()
    NC, NS, L = info.num_cores, info.num_subcores, info.num_lanes
    NW = NC * NS
    assert B % (8 * NW) == 0 and D % L == 0
    b_per_w = B // NW
    n_rounds = b_per_w // _ROUND

    mesh = plsc.VectorSubcoreMesh(core_axis_name="c", subcore_axis_name="s")

    @functools.partial(
        pl.kernel,
        mesh=mesh,
        out_type=jax.ShapeDtypeStruct((B * D,), jnp.float32),
        compiler_params=pltpu.CompilerParams(use_tc_tiling_on_sc=True),
        scratch_types=[
            pltpu.VMEM((b_per_w,), jnp.int32),
            pltpu.VMEM((b_per_w,), jnp.int32),
            pltpu.VMEM((b_per_w,), jnp.int32),
            pltpu.VMEM((b_per_w,), jnp.int32),
            pltpu.VMEM((_ROUND, TR, D), jnp.float32),
            pltpu.VMEM((2, _ROUND * D), jnp.float32),
            pltpu.SemaphoreType.DMA,
            pltpu.SemaphoreType.DMA,
        ],
    )
    def gather_kernel(
        l0_hbm, l1_hbm, w3_hbm, out_hbm,
        l0_v, l1_v, tid_v, sub_v, tiles_v, stage_v, gsem, osem,
    ):
        wid = lax.axis_index("s") * NC + lax.axis_index("c")
        base = wid * b_per_w
        pltpu.sync_copy(l0_hbm.at[pl.ds(base, b_per_w)], l0_v)
        pltpu.sync_copy(l1_hbm.at[pl.ds(base, b_per_w)], l1_v)
        for c in range(b_per_w // L):
            r = l0_v[pl.ds(c * L, L)] * num_objs + l1_v[pl.ds(c * L, L)]
            tid_v[pl.ds(c * L, L)] = r >> 3
            sub_v[pl.ds(c * L, L)] = r & 7

        def round_body(k, _):
            sl = lax.rem(k, 2)

            def fire_g(g, _2):
                tvec = tid_v[pl.ds(k * _ROUND + g * L, L)]
                for j in range(L):
                    pltpu.async_copy(
                        w3_hbm.at[tvec[j]], tiles_v.at[g * L + j], gsem
                    )
                return _2

            lax.fori_loop(0, _ROUND // L, fire_g, None)
            pltpu.make_async_copy(
                w3_hbm.at[pl.ds(0, _ROUND)], tiles_v, gsem
            ).wait()

            @pl.when(k >= 2)
            def _reclaim():
                pltpu.make_async_copy(
                    out_hbm.at[pl.ds(0, _ROUND * D)], stage_v.at[sl], osem
                ).wait()

            def ext_g(g, _2):
                svec = sub_v[pl.ds(k * _ROUND + g * L, L)]
                for j in range(L):
                    slot = g * L + j
                    s = svec[j]
                    for c in range(D // L):
                        stage_v[sl, pl.ds(slot * D + c * L, L)] = (
                            tiles_v[slot, s, pl.ds(c * L, L)]
                        )
                return _2

            lax.fori_loop(0, _ROUND // L, ext_g, None)
            pltpu.async_copy(
                stage_v.at[sl],
                out_hbm.at[pl.ds((base + k * _ROUND) * D, _ROUND * D)],
                osem,
            )
            return _

        lax.fori_loop(0, n_rounds, round_body, None)
        for t in range(2):
            pltpu.make_async_copy(
                out_hbm.at[pl.ds(0, _ROUND * D)], stage_v.at[t], osem
            ).wait()

    return gather_kernel


def kernel(labels, W):
    B = labels.shape[0]
    V, D = W.shape
    num_objs = math.isqrt(V)
    w3 = W.reshape(V // 8, 8, D)
    l0 = labels[:, 0].astype(jnp.int32)
    l1 = labels[:, 1].astype(jnp.int32)
    flat = _make_gather(B, V // 8, 8, D, num_objs)(l0, l1, w3)
    return flat.reshape(B, D)


# speedup vs baseline: 2.3189x; 1.0152x over previous
import functools
import math

import jax
import jax.numpy as jnp
from jax import lax
from jax.experimental import pallas as pl
from jax.experimental.pallas import tpu as pltpu
from jax.experimental.pallas import tpu_sc as plsc

_ROUND = 32


@functools.lru_cache(maxsize=None)
def _make_gather(B, NT, TR, D, num_objs):
    info = plsc.get_sparse_core_info()
    NC, NS, L = info.num_cores, info.num_subcores, info.num_lanes
    NW = NC * NS
    assert B % (8 * NW) == 0 and D % L == 0
    b_per_w = B // NW
    n_rounds = b_per_w // _ROUND
    n_pairs = n_rounds // 2

    mesh = plsc.VectorSubcoreMesh(core_axis_name="c", subcore_axis_name="s")

    @functools.partial(
        pl.kernel,
        mesh=mesh,
        out_type=jax.ShapeDtypeStruct((B * D,), jnp.float32),
        compiler_params=pltpu.CompilerParams(use_tc_tiling_on_sc=True),
        scratch_types=[
            pltpu.VMEM((b_per_w,), jnp.int32),
            pltpu.VMEM((b_per_w,), jnp.int32),
            pltpu.VMEM((b_per_w,), jnp.int32),
            pltpu.VMEM((b_per_w,), jnp.int32),
            pltpu.VMEM((2, _ROUND, TR, D), jnp.float32),
            pltpu.VMEM((2, _ROUND * D), jnp.float32),
            pltpu.SemaphoreType.DMA,
            pltpu.SemaphoreType.DMA,
            pltpu.SemaphoreType.DMA,
        ],
    )
    def gather_kernel(
        l0_hbm, l1_hbm, w3_hbm, out_hbm,
        l0_v, l1_v, tid_v, sub_v, tiles_v, stage_v, gsem0, gsem1, osem,
    ):
        wid = lax.axis_index("s") * NC + lax.axis_index("c")
        base = wid * b_per_w
        pltpu.sync_copy(l0_hbm.at[pl.ds(base, b_per_w)], l0_v)
        pltpu.sync_copy(l1_hbm.at[pl.ds(base, b_per_w)], l1_v)
        for c in range(b_per_w // L):
            r = l0_v[pl.ds(c * L, L)] * num_objs + l1_v[pl.ds(c * L, L)]
            tid_v[pl.ds(c * L, L)] = r >> 3
            sub_v[pl.ds(c * L, L)] = r & 7

        def fire(k, sl, sem):
            def fire_g(g, _2):
                tvec = tid_v[pl.ds(k * _ROUND + g * L, L)]
                for j in range(L):
                    pltpu.async_copy(
                        w3_hbm.at[tvec[j]], tiles_v.at[sl, g * L + j], sem
                    )
                return _2

            lax.fori_loop(0, _ROUND // L, fire_g, None)

        def drain_tiles(sem):
            pltpu.make_async_copy(
                w3_hbm.at[pl.ds(0, _ROUND)], tiles_v.at[0], sem
            ).wait()

        def reclaim_stage(sl):
            pltpu.make_async_copy(
                out_hbm.at[pl.ds(0, _ROUND * D)], stage_v.at[sl], osem
            ).wait()

        def extract_and_put(k, sl):
            def ext_g(g, _2):
                svec = sub_v[pl.ds(k * _ROUND + g * L, L)]
                for j in range(L):
                    slot = g * L + j
                    s = svec[j]
                    for c in range(D // L):
                        stage_v[sl, pl.ds(slot * D + c * L, L)] = (
                            tiles_v[sl, slot, s, pl.ds(c * L, L)]
                        )
                return _2

            lax.fori_loop(0, _ROUND // L, ext_g, None)
            pltpu.async_copy(
                stage_v.at[sl],
                out_hbm.at[pl.ds((base + k * _ROUND) * D, _ROUND * D)],
                osem,
            )

        fire(0, 0, gsem0)

        def pair_body(kk, _):
            k0 = 2 * kk
            fire(k0 + 1, 1, gsem1)
            drain_tiles(gsem0)

            @pl.when(kk >= 1)
            def _r0():
                reclaim_stage(0)

            extract_and_put(k0, 0)

            @pl.when(kk + 1 < n_pairs)
            def _f0():
                fire(k0 + 2, 0, gsem0)

            drain_tiles(gsem1)

            @pl.when(kk >= 1)
            def _r1():
                reclaim_stage(1)

            extract_and_put(k0 + 1, 1)
            return _

        lax.fori_loop(0, n_pairs, pair_body, None)
        reclaim_stage(0)
        reclaim_stage(1)

    return gather_kernel


def kernel(labels, W):
    B = labels.shape[0]
    V, D = W.shape
    num_objs = math.isqrt(V)
    w3 = W.reshape(V // 8, 8, D)
    l0 = labels[:, 0].astype(jnp.int32)
    l1 = labels[:, 1].astype(jnp.int32)
    flat = _make_gather(B, V // 8, 8, D, num_objs)(l0, l1, w3)
    return flat.reshape(B, D)


# speedup vs baseline: 2.3801x; 1.0264x over previous
import functools
import math

import jax
import jax.numpy as jnp
from jax import lax
from jax.experimental import pallas as pl
from jax.experimental.pallas import tpu as pltpu
from jax.experimental.pallas import tpu_sc as plsc

_ROUND = 32


@functools.lru_cache(maxsize=None)
def _make_gather(B, NT, TR, D, num_objs):
    info = plsc.get_sparse_core_info()
    NC, NS, L = info.num_cores, info.num_subcores, info.num_lanes
    NW = NC * NS
    assert B % (8 * NW) == 0 and D % L == 0
    b_per_w = B // NW
    n_rounds = b_per_w // _ROUND
    n_pairs = n_rounds // 2

    mesh = plsc.VectorSubcoreMesh(core_axis_name="c", subcore_axis_name="s")

    @functools.partial(
        pl.kernel,
        mesh=mesh,
        out_type=jax.ShapeDtypeStruct((B, D), jnp.float32),
        compiler_params=pltpu.CompilerParams(use_tc_tiling_on_sc=True),
        scratch_types=[
            pltpu.VMEM((b_per_w,), jnp.int32),
            pltpu.VMEM((b_per_w,), jnp.int32),
            pltpu.VMEM((b_per_w,), jnp.int32),
            pltpu.VMEM((b_per_w,), jnp.int32),
            pltpu.VMEM((2, _ROUND, TR, D), jnp.float32),
            pltpu.VMEM((2, _ROUND, D), jnp.float32),
            pltpu.SemaphoreType.DMA,
            pltpu.SemaphoreType.DMA,
            pltpu.SemaphoreType.DMA,
        ],
    )
    def gather_kernel(
        l0_hbm, l1_hbm, w3_hbm, out_hbm,
        l0_v, l1_v, tid_v, sub_v, tiles_v, stage_v, gsem0, gsem1, osem,
    ):
        wid = lax.axis_index("s") * NC + lax.axis_index("c")
        base = wid * b_per_w
        pltpu.sync_copy(l0_hbm.at[pl.ds(base, b_per_w)], l0_v)
        pltpu.sync_copy(l1_hbm.at[pl.ds(base, b_per_w)], l1_v)
        for c in range(b_per_w // L):
            r = l0_v[pl.ds(c * L, L)] * num_objs + l1_v[pl.ds(c * L, L)]
            tid_v[pl.ds(c * L, L)] = r >> 3
            sub_v[pl.ds(c * L, L)] = r & 7

        def fire(k, sl, sem):
            def fire_g(g, _2):
                tvec = tid_v[pl.ds(k * _ROUND + g * L, L)]
                for j in range(L):
                    pltpu.async_copy(
                        w3_hbm.at[tvec[j]], tiles_v.at[sl, g * L + j], sem
                    )
                return _2

            lax.fori_loop(0, _ROUND // L, fire_g, None)

        def drain_tiles(sem):
            pltpu.make_async_copy(
                w3_hbm.at[pl.ds(0, _ROUND)], tiles_v.at[0], sem
            ).wait()

        def reclaim_stage(sl):
            pltpu.make_async_copy(
                out_hbm.at[pl.ds(0, _ROUND), :], stage_v.at[sl], osem
            ).wait()

        def extract_and_put(k, sl):
            def ext_g(g, _2):
                svec = sub_v[pl.ds(k * _ROUND + g * L, L)]
                for j in range(L):
                    slot = g * L + j
                    s = svec[j]
                    for c in range(D // L):
                        stage_v[sl, slot, pl.ds(c * L, L)] = (
                            tiles_v[sl, slot, s, pl.ds(c * L, L)]
                        )
                return _2

            lax.fori_loop(0, _ROUND // L, ext_g, None)
            pltpu.async_copy(
                stage_v.at[sl],
                out_hbm.at[pl.ds(base + k * _ROUND, _ROUND), :],
                osem,
            )

        fire(0, 0, gsem0)

        def pair_body(kk, _):
            k0 = 2 * kk
            fire(k0 + 1, 1, gsem1)
            drain_tiles(gsem0)

            @pl.when(kk >= 1)
            def _r0():
                reclaim_stage(0)

            extract_and_put(k0, 0)

            @pl.when(kk + 1 < n_pairs)
            def _f0():
                fire(k0 + 2, 0, gsem0)

            drain_tiles(gsem1)

            @pl.when(kk >= 1)
            def _r1():
                reclaim_stage(1)

            extract_and_put(k0 + 1, 1)
            return _

        lax.fori_loop(0, n_pairs, pair_body, None)
        reclaim_stage(0)
        reclaim_stage(1)

    return gather_kernel


def kernel(labels, W):
    B = labels.shape[0]
    V, D = W.shape
    num_objs = math.isqrt(V)
    w3 = W.reshape(V // 8, 8, D)
    l0 = labels[:, 0].astype(jnp.int32)
    l1 = labels[:, 1].astype(jnp.int32)
    return _make_gather(B, V // 8, 8, D, num_objs)(l0, l1, w3)


# speedup vs baseline: 2.3858x; 1.0024x over previous
import functools
import math

import jax
import jax.numpy as jnp
from jax import lax
from jax.experimental import pallas as pl
from jax.experimental.pallas import tpu as pltpu
from jax.experimental.pallas import tpu_sc as plsc

_ROUND = 32


@functools.lru_cache(maxsize=None)
def _make_gather(B, NT, TR, D, num_objs):
    info = plsc.get_sparse_core_info()
    NC, NS, L = info.num_cores, info.num_subcores, info.num_lanes
    NW = NC * NS
    assert B % (8 * NW) == 0 and D % L == 0
    b_per_w = B // NW
    n_rounds = b_per_w // _ROUND
    n_pairs = n_rounds // 2

    mesh = plsc.VectorSubcoreMesh(core_axis_name="c", subcore_axis_name="s")

    @functools.partial(
        pl.kernel,
        mesh=mesh,
        out_type=jax.ShapeDtypeStruct((B, D), jnp.float32),
        compiler_params=pltpu.CompilerParams(use_tc_tiling_on_sc=True),
        scratch_types=[
            pltpu.VMEM((b_per_w,), jnp.int32),
            pltpu.VMEM((b_per_w,), jnp.int32),
            pltpu.VMEM((2, _ROUND, TR, D), jnp.float32),
            pltpu.VMEM((2, _ROUND, D), jnp.float32),
            pltpu.SemaphoreType.DMA,
            pltpu.SemaphoreType.DMA,
            pltpu.SemaphoreType.DMA,
        ],
    )
    def gather_kernel(
        tid_hbm, sub_hbm, w3_hbm, out_hbm,
        tid_v, sub_v, tiles_v, stage_v, gsem0, gsem1, osem,
    ):
        wid = lax.axis_index("s") * NC + lax.axis_index("c")
        base = wid * b_per_w
        pltpu.sync_copy(tid_hbm.at[pl.ds(base, b_per_w)], tid_v)
        pltpu.sync_copy(sub_hbm.at[pl.ds(base, b_per_w)], sub_v)

        def fire(k, sl, sem):
            def fire_g(g, _2):
                tvec = tid_v[pl.ds(k * _ROUND + g * L, L)]
                for j in range(L):
                    pltpu.async_copy(
                        w3_hbm.at[tvec[j]], tiles_v.at[sl, g * L + j], sem
                    )
                return _2

            lax.fori_loop(0, _ROUND // L, fire_g, None)

        def drain_tiles(sem):
            pltpu.make_async_copy(
                w3_hbm.at[pl.ds(0, _ROUND)], tiles_v.at[0], sem
            ).wait()

        def reclaim_stage(sl):
            pltpu.make_async_copy(
                out_hbm.at[pl.ds(0, _ROUND), :], stage_v.at[sl], osem
            ).wait()

        def extract_and_put(k, sl):
            def ext_g(g, _2):
                svec = sub_v[pl.ds(k * _ROUND + g * L, L)]
                for j in range(L):
                    slot = g * L + j
                    s = svec[j]
                    for c in range(D // L):
                        stage_v[sl, slot, pl.ds(c * L, L)] = (
                            tiles_v[sl, slot, s, pl.ds(c * L, L)]
                        )
                return _2

            lax.fori_loop(0, _ROUND // L, ext_g, None)
            pltpu.async_copy(
                stage_v.at[sl],
                out_hbm.at[pl.ds(base + k * _ROUND, _ROUND), :],
                osem,
            )

        fire(0, 0, gsem0)

        def pair_body(kk, _):
            k0 = 2 * kk
            fire(k0 + 1, 1, gsem1)
            drain_tiles(gsem0)

            @pl.when(kk >= 1)
            def _r0():
                reclaim_stage(0)

            extract_and_put(k0, 0)

            @pl.when(kk + 1 < n_pairs)
            def _f0():
                fire(k0 + 2, 0, gsem0)

            drain_tiles(gsem1)

            @pl.when(kk >= 1)
            def _r1():
                reclaim_stage(1)

            extract_and_put(k0 + 1, 1)
            return _

        lax.fori_loop(0, n_pairs, pair_body, None)
        reclaim_stage(0)
        reclaim_stage(1)

    return gather_kernel


def kernel(labels, W):
    B = labels.shape[0]
    V, D = W.shape
    num_objs = math.isqrt(V)
    w3 = W.reshape(V // 8, 8, D)
    r = labels[:, 0].astype(jnp.int32) * num_objs + labels[:, 1].astype(jnp.int32)
    return _make_gather(B, V // 8, 8, D, num_objs)(r >> 3, r & 7, w3)
